# Initial kernel scaffold; baseline (speedup 1.0000x reference)
import jax
import jax.numpy as jnp
from jax.experimental import pallas as pl


def kernel(hidden_states, W, b):
    raise NotImplementedError("write your pallas kernel here")


# speedup vs baseline: 3.3915x; 3.3915x over previous
import jax
import jax.numpy as jnp
from jax.experimental import pallas as pl

TOP_K = 2
NUM_EXPERTS = 64
HIDDEN = 2048
TOKENS = 8192
BLOCK_T = 512


def _router_block(x_ref, w_ref, b_ref, scores_ref, idx_ref):
    x = x_ref[...]
    w = w_ref[...]
    logits = jax.lax.dot_general(
        x, w, (((1,), (1,)), ((), ())), preferred_element_type=jnp.float32)
    logits = logits + b_ref[...]

    eidx = jax.lax.broadcasted_iota(jnp.int32, logits.shape, 1)
    m1 = jnp.max(logits, axis=1, keepdims=True)
    i1 = jnp.min(jnp.where(logits == m1, eidx, NUM_EXPERTS), axis=1,
                 keepdims=True)
    masked = jnp.where(eidx == i1, -jnp.inf, logits)
    m2 = jnp.max(masked, axis=1, keepdims=True)
    i2 = jnp.min(jnp.where(masked == m2, eidx, NUM_EXPERTS), axis=1,
                 keepdims=True)

    d = jnp.exp(m2 - m1)
    p1 = 1.0 / (1.0 + d)
    p2 = d * p1

    zeros = jnp.zeros_like(logits)
    scores = (jnp.where(eidx == i1, p1, zeros)
              + jnp.where(eidx == i2, p2, zeros))
    scores_ref[...] = scores
    idx_ref[...] = jnp.concatenate([i1, i2], axis=1)


def kernel(hidden_states, W, b):
    x = hidden_states.reshape(-1, HIDDEN)
    T = x.shape[0]
    grid = (T // BLOCK_T,)
    scores, idx = pl.pallas_call(
        _router_block,
        grid=grid,
        in_specs=[
            pl.BlockSpec((BLOCK_T, HIDDEN), lambda i: (i, 0)),
            pl.BlockSpec((NUM_EXPERTS, HIDDEN), lambda i: (0, 0)),
            pl.BlockSpec((1, NUM_EXPERTS), lambda i: (0, 0)),
        ],
        out_specs=[
            pl.BlockSpec((BLOCK_T, NUM_EXPERTS), lambda i: (i, 0)),
            pl.BlockSpec((BLOCK_T, TOP_K), lambda i: (i, 0)),
        ],
        out_shape=[
            jax.ShapeDtypeStruct((T, NUM_EXPERTS), jnp.float32),
            jax.ShapeDtypeStruct((T, TOP_K), jnp.int32),
        ],
    )(x, W, b.reshape(1, NUM_EXPERTS))
    return scores, idx
